# baseline (speedup 1.0000x reference)
import functools

import jax
import jax.numpy as jnp
from jax import lax
from jax.experimental import pallas as pl
from jax.experimental.pallas import tpu as pltpu
from jax.experimental.pallas import tpu_sc as plsc

EMB = 128
NC, NS = 2, 16
NW = NC * NS


def _table_body(w0_ref, w1_ref, w2_ref, t_ref):
    c = lax.broadcasted_iota(jnp.int32, (64, 1), 0)
    i0, r = c // 12, c % 12
    i1, i2 = r // 2, r % 2

    def oh(idx):
        return (idx == lax.broadcasted_iota(jnp.int32, (1, 8), 1)
                ).astype(jnp.float32)

    t_ref[...] = (
        jnp.dot(oh(i0), w0_ref[...], preferred_element_type=jnp.float32)
        + jnp.dot(oh(i1), w1_ref[...], preferred_element_type=jnp.float32)
        + jnp.dot(oh(i2), w2_ref[...], preferred_element_type=jnp.float32))


def _build_table(W0, W1, W2):
    def pad8(w):
        return jnp.zeros((8, EMB), jnp.float32).at[:w.shape[0]].set(w)

    return pl.pallas_call(
        _table_body,
        out_shape=jax.ShapeDtypeStruct((64, EMB), jnp.float32),
    )(pad8(W0), pad8(W1), pad8(W2))


def _make_sc_kernel(E):
    per_w = E // NW
    chunk = 80
    nbuf = 5
    g16 = chunk // 16
    n_rounds = per_w // (nbuf * chunk)
    mesh = plsc.VectorSubcoreMesh(core_axis_name="c", subcore_axis_name="s")

    @functools.partial(
        pl.kernel, mesh=mesh,
        compiler_params=pltpu.CompilerParams(needs_layout_passes=False),
        out_type=jax.ShapeDtypeStruct((E * EMB,), jnp.float32),
        scratch_types=[
            pltpu.VMEM((64 * EMB,), jnp.float32),
            pltpu.VMEM((per_w,), jnp.int32),
            pltpu.VMEM((per_w,), jnp.int32),
            pltpu.VMEM((per_w,), jnp.int32),
            pltpu.VMEM((per_w,), jnp.int32),
        ] + [pltpu.VMEM((chunk * EMB,), jnp.float32) for _ in range(nbuf)]
          + [pltpu.SemaphoreType.DMA for _ in range(nbuf)])
    def k(attr_hbm, t_hbm, out_hbm, t_v, a0_v, a1_v, a2_v, codes_v,
          *bufs_and_sems):
        rows = bufs_and_sems[:nbuf]
        sems = bufs_and_sems[nbuf:]
        wid = lax.axis_index("s") * NC + lax.axis_index("c")
        tile_base = wid * per_w

        pltpu.sync_copy(t_hbm, t_v)
        for c, av in enumerate((a0_v, a1_v, a2_v)):
            pltpu.sync_copy(attr_hbm.at[pl.ds(c * E + tile_base, per_w)], av)

        def code_body(j, carry):
            s = pl.ds(j * 16, 16)
            codes_v[s] = (a0_v[s] * 12 + a1_v[s] * 2 + a2_v[s]) * EMB
            return carry

        lax.fori_loop(0, per_w // 16, code_body, 0, unroll=5)

        lanes128 = lax.iota(jnp.int32, 16) * EMB

        def round_body(r, carry):
            for b in range(nbuf):
                base = (r * nbuf + b) * chunk
                rb, sb = rows[b], sems[b]

                @pl.when(r >= 1)
                def _():
                    pltpu.make_async_copy(
                        rb, out_hbm.at[pl.ds(0, chunk * EMB)], sb).wait()

                def copy_group(u, carry2):
                    ridx = codes_v[pl.ds(base + u * 16, 16)]
                    widx = lanes128 + u * (16 * EMB)
                    for p in range(EMB):
                        vals = plsc.load_gather(t_v, [ridx + p])
                        plsc.store_scatter(rb, [widx + p], vals)
                    return carry2

                lax.fori_loop(0, g16, copy_group, 0)
                pltpu.async_copy(
                    rb, out_hbm.at[pl.ds((tile_base + base) * EMB,
                                         chunk * EMB)], sb)
            return carry

        lax.fori_loop(0, n_rounds, round_body, 0)
        for b in range(nbuf):
            pltpu.make_async_copy(
                rows[b], out_hbm.at[pl.ds(0, chunk * EMB)], sems[b]).wait()

    return k


def kernel(edge_attr, W0, W1, W2):
    E = edge_attr.shape[0]
    attr = edge_attr.astype(jnp.int32).T.reshape(-1)
    t = _build_table(W0, W1, W2).reshape(-1)
    return _make_sc_kernel(E)(attr, t).reshape(E, EMB)


# speedup vs baseline: 14.5610x; 14.5610x over previous
import functools

import jax
import jax.numpy as jnp
from jax import lax
from jax.experimental import pallas as pl
from jax.experimental.pallas import tpu as pltpu
from jax.experimental.pallas import tpu_sc as plsc

EMB = 128
NC, NS = 2, 16
NW = NC * NS
CH = 80
GROUPS = CH // 16


def _table_body(w0_ref, w1_ref, w2_ref, t_ref):
    c = lax.broadcasted_iota(jnp.int32, (64, 1), 0)
    i0, r = c // 12, c % 12
    i1, i2 = r // 2, r % 2

    def oh(idx):
        return (idx == lax.broadcasted_iota(jnp.int32, (1, 8), 1)
                ).astype(jnp.float32)

    t_ref[...] = (
        jnp.dot(oh(i0), w0_ref[...], preferred_element_type=jnp.float32)
        + jnp.dot(oh(i1), w1_ref[...], preferred_element_type=jnp.float32)
        + jnp.dot(oh(i2), w2_ref[...], preferred_element_type=jnp.float32))


def _build_table(W0, W1, W2):
    def pad8(w):
        return jnp.zeros((8, EMB), jnp.float32).at[:w.shape[0]].set(w)

    t = pl.pallas_call(
        _table_body,
        out_shape=jax.ShapeDtypeStruct((64, EMB), jnp.float32),
    )(pad8(W0), pad8(W1), pad8(W2))
    return jnp.broadcast_to(t, (NW, 64, EMB)).reshape(NW * 64, EMB)


def _make_sc_kernel(E):
    per_w = E // NW
    chunk = 200
    sub = 40
    nsub = chunk // sub
    groups16 = per_w // 16
    n_groups = per_w // (2 * chunk)
    mesh = plsc.VectorSubcoreMesh(core_axis_name="c", subcore_axis_name="s")

    @functools.partial(
        pl.kernel, mesh=mesh,
        out_type=jax.ShapeDtypeStruct((E, EMB), jnp.float32),
        scratch_types=[
            pltpu.VMEM_SHARED((64, EMB), jnp.float32),
            pltpu.VMEM((per_w,), jnp.int32),
            pltpu.VMEM((per_w,), jnp.int32),
            pltpu.VMEM((per_w,), jnp.int32),
            pltpu.VMEM((per_w,), jnp.int32),
            pltpu.VMEM((chunk, EMB), jnp.float32),
            pltpu.VMEM((chunk, EMB), jnp.float32),
            pltpu.SemaphoreType.DMA,
            pltpu.SemaphoreType.DMA,
            pltpu.SemaphoreType.DMA,
        ])
    def k(attr_hbm, t_hbm, out_hbm, t_s, a0_v, a1_v, a2_v, codes_v, rows0,
          rows1, sem_g, sem_o0, sem_o1):
        wid = lax.axis_index("s") * NC + lax.axis_index("c")
        tile_base = wid * per_w
        @pl.when(lax.axis_index("s") == 0)
        def _():
            pltpu.sync_copy(t_hbm.at[pl.ds(0, 64)], t_s)
        plsc.subcore_barrier()

        for c, av in enumerate((a0_v, a1_v, a2_v)):
            pltpu.sync_copy(attr_hbm.at[pl.ds(c * E + tile_base, per_w)], av)

        code_off = wid * 64

        def code_body(j, carry):
            s = pl.ds(j * 16, 16)
            codes_v[s] = a0_v[s] * 12 + a1_v[s] * 2 + a2_v[s]
            return carry

        lax.fori_loop(0, groups16, code_body, 0, unroll=5)

        rows = (rows0, rows1)
        sems_o = (sem_o0, sem_o1)

        def chunk_body(g, carry):
            for b in range(2):
                base = (2 * g + b) * chunk
                rb, so = rows[b], sems_o[b]

                @pl.when(g >= 1)
                def _():
                    for j in range(nsub):
                        pltpu.make_async_copy(
                            rb.at[pl.ds(j * sub, sub)],
                            out_hbm.at[pl.ds(tile_base + base + j * sub, sub)],
                            so).wait()

                hs = [pltpu.async_copy(
                          t_s.at[codes_v.at[pl.ds(base + j * sub, sub)]],
                          rb.at[pl.ds(j * sub, sub)], sem_g)
                      for j in range(nsub)]
                for h in hs:
                    h.wait()
                for j in range(nsub):
                    pltpu.async_copy(
                        rb.at[pl.ds(j * sub, sub)],
                        out_hbm.at[pl.ds(tile_base + base + j * sub, sub)],
                        so)
            return carry

        lax.fori_loop(0, n_groups, chunk_body, 0)

        for b in range(2):
            base = (2 * (n_groups - 1) + b) * chunk
            for j in range(nsub):
                pltpu.make_async_copy(
                    rows[b].at[pl.ds(j * sub, sub)],
                    out_hbm.at[pl.ds(tile_base + base + j * sub, sub)],
                    sems_o[b]).wait()

    return k


def kernel(edge_attr, W0, W1, W2):
    E = edge_attr.shape[0]
    attr = edge_attr.astype(jnp.int32).T.reshape(-1)
    t = _build_table(W0, W1, W2)
    return _make_sc_kernel(E)(attr, t)


# speedup vs baseline: 14.9282x; 1.0252x over previous
import functools

import jax
import jax.numpy as jnp
from jax import lax
from jax.experimental import pallas as pl
from jax.experimental.pallas import tpu as pltpu
from jax.experimental.pallas import tpu_sc as plsc

EMB = 128
NC, NS = 2, 16
NW = NC * NS
CH = 80
GROUPS = CH // 16


def _table_body(w0_ref, w1_ref, w2_ref, t_ref):
    c = lax.broadcasted_iota(jnp.int32, (64, 1), 0)
    i0, r = c // 12, c % 12
    i1, i2 = r // 2, r % 2

    def pick(idx, w_ref):
        acc = jnp.zeros((64, EMB), jnp.float32)
        for row in range(w_ref.shape[0]):
            acc = acc + (idx == row).astype(jnp.float32) * w_ref[row:row + 1, :]
        return acc

    t_ref[...] = (pick(i0, w0_ref) + pick(i1, w1_ref) + pick(i2, w2_ref))


def _build_table(W0, W1, W2):
    def pad8(w):
        return jnp.zeros((8, EMB), jnp.float32).at[:w.shape[0]].set(w)

    return pl.pallas_call(
        _table_body,
        out_shape=jax.ShapeDtypeStruct((64, EMB), jnp.float32),
    )(pad8(W0), pad8(W1), pad8(W2))


def _make_sc_kernel(E):
    per_w = E // NW
    chunk = 200
    sub = 40
    nsub = chunk // sub
    groups16 = per_w // 16
    n_groups = per_w // (2 * chunk)
    mesh = plsc.VectorSubcoreMesh(core_axis_name="c", subcore_axis_name="s")

    @functools.partial(
        pl.kernel, mesh=mesh,
        out_type=jax.ShapeDtypeStruct((E, EMB), jnp.float32),
        scratch_types=[
            pltpu.VMEM_SHARED((64, EMB), jnp.float32),
            pltpu.VMEM((per_w,), jnp.int32),
            pltpu.VMEM((per_w,), jnp.int32),
            pltpu.VMEM((per_w,), jnp.int32),
            pltpu.VMEM((per_w,), jnp.int32),
            pltpu.VMEM((chunk, EMB), jnp.float32),
            pltpu.VMEM((chunk, EMB), jnp.float32),
            pltpu.SemaphoreType.DMA,
            pltpu.SemaphoreType.DMA,
            pltpu.SemaphoreType.DMA,
        ])
    def k(attr_hbm, t_hbm, out_hbm, t_s, a0_v, a1_v, a2_v, codes_v, rows0,
          rows1, sem_g, sem_o0, sem_o1):
        wid = lax.axis_index("s") * NC + lax.axis_index("c")
        tile_base = wid * per_w
        @pl.when(lax.axis_index("s") == 0)
        def _():
            pltpu.sync_copy(t_hbm, t_s)
        plsc.subcore_barrier()

        for c, av in enumerate((a0_v, a1_v, a2_v)):
            pltpu.sync_copy(attr_hbm.at[pl.ds(c * E + tile_base, per_w)], av)

        def code_body(j, carry):
            s = pl.ds(j * 16, 16)
            codes_v[s] = a0_v[s] * 12 + a1_v[s] * 2 + a2_v[s]
            return carry

        lax.fori_loop(0, groups16, code_body, 0, unroll=5)

        rows = (rows0, rows1)
        sems_o = (sem_o0, sem_o1)

        def chunk_body(g, carry):
            for b in range(2):
                base = (2 * g + b) * chunk
                rb, so = rows[b], sems_o[b]

                @pl.when(g >= 1)
                def _():
                    pltpu.make_async_copy(
                        rb, out_hbm.at[pl.ds(tile_base + base, chunk)],
                        so).wait()

                hs = [pltpu.async_copy(
                          t_s.at[codes_v.at[pl.ds(base + j * sub, sub)]],
                          rb.at[pl.ds(j * sub, sub)], sem_g)
                      for j in range(nsub)]
                for h in hs:
                    h.wait()
                pltpu.async_copy(
                    rb, out_hbm.at[pl.ds(tile_base + base, chunk)], so)
            return carry

        lax.fori_loop(0, n_groups, chunk_body, 0)

        for b in range(2):
            base = (2 * (n_groups - 1) + b) * chunk
            pltpu.make_async_copy(
                rows[b], out_hbm.at[pl.ds(tile_base + base, chunk)],
                sems_o[b]).wait()

    return k


def kernel(edge_attr, W0, W1, W2):
    E = edge_attr.shape[0]
    attr = edge_attr.astype(jnp.int32).T.reshape(-1)
    t = _build_table(W0, W1, W2)
    return _make_sc_kernel(E)(attr, t)


# speedup vs baseline: 15.1358x; 1.0139x over previous
import functools

import jax
import jax.numpy as jnp
from jax import lax
from jax.experimental import pallas as pl
from jax.experimental.pallas import tpu as pltpu
from jax.experimental.pallas import tpu_sc as plsc

EMB = 128
NC, NS = 2, 16
NW = NC * NS
CH = 80
GROUPS = CH // 16


def _table_body(w0_ref, w1_ref, w2_ref, t_ref):
    c = lax.broadcasted_iota(jnp.int32, (64, 1), 0)
    i0, r = c // 12, c % 12
    i1, i2 = r // 2, r % 2

    def pick(idx, w_ref):
        acc = jnp.zeros((64, EMB), jnp.float32)
        for row in range(w_ref.shape[0]):
            acc = acc + (idx == row).astype(jnp.float32) * w_ref[row:row + 1, :]
        return acc

    t_ref[...] = (pick(i0, w0_ref) + pick(i1, w1_ref) + pick(i2, w2_ref))


def _build_table(W0, W1, W2):
    def pad8(w):
        return jnp.zeros((8, EMB), jnp.float32).at[:w.shape[0]].set(w)

    return pl.pallas_call(
        _table_body,
        out_shape=jax.ShapeDtypeStruct((64, EMB), jnp.float32),
    )(pad8(W0), pad8(W1), pad8(W2))


def _make_sc_kernel(E):
    per_w = E // NW
    chunk = 200
    sub = 40
    nsub = chunk // sub
    groups16 = per_w // 16
    n_groups = per_w // (2 * chunk)
    mesh = plsc.VectorSubcoreMesh(core_axis_name="c", subcore_axis_name="s")

    @functools.partial(
        pl.kernel, mesh=mesh,
        out_type=jax.ShapeDtypeStruct((E, EMB), jnp.float32),
        scratch_types=[
            pltpu.VMEM_SHARED((64, EMB), jnp.float32),
            pltpu.VMEM((per_w,), jnp.int32),
            pltpu.VMEM((per_w,), jnp.int32),
            pltpu.VMEM((per_w,), jnp.int32),
            pltpu.VMEM((per_w,), jnp.int32),
            pltpu.VMEM((chunk, EMB), jnp.float32),
            pltpu.VMEM((chunk, EMB), jnp.float32),
            pltpu.SemaphoreType.DMA,
            pltpu.SemaphoreType.DMA,
            pltpu.SemaphoreType.DMA,
        ])
    def k(attr_hbm, t_hbm, out_hbm, t_s, a0_v, a1_v, a2_v, codes_v, rows0,
          rows1, sem_g, sem_o0, sem_o1):
        wid = lax.axis_index("s") * NC + lax.axis_index("c")
        tile_base = wid * per_w
        @pl.when(lax.axis_index("s") == 0)
        def _():
            pltpu.sync_copy(t_hbm, t_s)
        plsc.subcore_barrier()

        hs_a = [pltpu.async_copy(
                    attr_hbm.at[pl.ds(c * E + tile_base, per_w)], av, sem_g)
                for c, av in enumerate((a0_v, a1_v, a2_v))]
        for h in hs_a:
            h.wait()

        def code_body(j, carry):
            s = pl.ds(j * 16, 16)
            codes_v[s] = a0_v[s] * 12 + a1_v[s] * 2 + a2_v[s]
            return carry

        lax.fori_loop(0, groups16, code_body, 0, unroll=5)

        rows = (rows0, rows1)
        sems_o = (sem_o0, sem_o1)

        def chunk_body(g, carry):
            for b in range(2):
                base = (2 * g + b) * chunk
                rb, so = rows[b], sems_o[b]

                @pl.when(g >= 1)
                def _():
                    pltpu.make_async_copy(
                        rb, out_hbm.at[pl.ds(tile_base + base, chunk)],
                        so).wait()

                hs = [pltpu.async_copy(
                          t_s.at[codes_v.at[pl.ds(base + j * sub, sub)]],
                          rb.at[pl.ds(j * sub, sub)], sem_g)
                      for j in range(nsub)]
                for h in hs:
                    h.wait()
                pltpu.async_copy(
                    rb, out_hbm.at[pl.ds(tile_base + base, chunk)], so)
            return carry

        lax.fori_loop(0, n_groups, chunk_body, 0)

        for b in range(2):
            base = (2 * (n_groups - 1) + b) * chunk
            pltpu.make_async_copy(
                rows[b], out_hbm.at[pl.ds(tile_base + base, chunk)],
                sems_o[b]).wait()

    return k


def kernel(edge_attr, W0, W1, W2):
    E = edge_attr.shape[0]
    attr = edge_attr.astype(jnp.int32).T.reshape(-1)
    t = _build_table(W0, W1, W2)
    return _make_sc_kernel(E)(attr, t)


# speedup vs baseline: 15.5210x; 1.0254x over previous
import functools

import jax
import jax.numpy as jnp
from jax import lax
from jax.experimental import pallas as pl
from jax.experimental.pallas import tpu as pltpu
from jax.experimental.pallas import tpu_sc as plsc

EMB = 128
NC, NS = 2, 16
NW = NC * NS
CH = 80
GROUPS = CH // 16


def _table_body(w0_ref, w1_ref, w2_ref, t_ref):
    c = lax.broadcasted_iota(jnp.int32, (64, 1), 0)
    i0, r = c // 12, c % 12
    i1, i2 = r // 2, r % 2

    def pick(idx, w_ref):
        acc = jnp.zeros((64, EMB), jnp.float32)
        for row in range(w_ref.shape[0]):
            acc = acc + (idx == row).astype(jnp.float32) * w_ref[row:row + 1, :]
        return acc

    t_ref[...] = (pick(i0, w0_ref) + pick(i1, w1_ref) + pick(i2, w2_ref))


def _build_table(W0, W1, W2):
    def pad8(w):
        return jnp.zeros((8, EMB), jnp.float32).at[:w.shape[0]].set(w)

    return pl.pallas_call(
        _table_body,
        out_shape=jax.ShapeDtypeStruct((64, EMB), jnp.float32),
    )(pad8(W0), pad8(W1), pad8(W2))


def _make_sc_kernel(E):
    per_w = E // NW
    chunk = 200
    sub = 40
    nsub = chunk // sub
    groups16 = per_w // 16
    n_groups = per_w // (2 * chunk)
    mesh = plsc.VectorSubcoreMesh(core_axis_name="c", subcore_axis_name="s")

    @functools.partial(
        pl.kernel, mesh=mesh,
        out_type=jax.ShapeDtypeStruct((E, EMB), jnp.float32),
        scratch_types=[
            pltpu.VMEM_SHARED((64, EMB), jnp.float32),
            pltpu.VMEM((per_w,), jnp.int32),
            pltpu.VMEM((per_w,), jnp.int32),
            pltpu.VMEM((per_w,), jnp.int32),
            pltpu.VMEM((per_w,), jnp.int32),
            pltpu.VMEM((chunk, EMB), jnp.float32),
            pltpu.VMEM((chunk, EMB), jnp.float32),
            pltpu.SemaphoreType.DMA,
            pltpu.SemaphoreType.DMA,
            pltpu.SemaphoreType.DMA,
        ])
    def k(attr_hbm, t_hbm, out_hbm, t_s, a0_v, a1_v, a2_v, codes_v, rows0,
          rows1, sem_g, sem_o0, sem_o1):
        wid = lax.axis_index("s") * NC + lax.axis_index("c")
        tile_base = wid * per_w
        @pl.when(lax.axis_index("s") == 0)
        def _():
            pltpu.sync_copy(t_hbm, t_s)
        plsc.subcore_barrier()

        hs_a = [pltpu.async_copy(
                    attr_hbm.at[pl.ds(c * E + tile_base, per_w)], av, sem_g)
                for c, av in enumerate((a0_v, a1_v, a2_v))]
        for h in hs_a:
            h.wait()

        rows = (rows0, rows1)
        sems_o = (sem_o0, sem_o1)

        def chunk_body(g, carry):
            pair_base = 2 * g * chunk
            for j in range(2 * chunk // 16):
                s = pl.ds(pair_base + j * 16, 16)
                codes_v[s] = a0_v[s] * 12 + a1_v[s] * 2 + a2_v[s]
            for b in range(2):
                base = (2 * g + b) * chunk
                rb, so = rows[b], sems_o[b]

                @pl.when(g >= 1)
                def _():
                    pltpu.make_async_copy(
                        rb, out_hbm.at[pl.ds(tile_base + base, chunk)],
                        so).wait()

                hs = [pltpu.async_copy(
                          t_s.at[codes_v.at[pl.ds(base + j * sub, sub)]],
                          rb.at[pl.ds(j * sub, sub)], sem_g)
                      for j in range(nsub)]
                for h in hs:
                    h.wait()
                pltpu.async_copy(
                    rb, out_hbm.at[pl.ds(tile_base + base, chunk)], so)
            return carry

        lax.fori_loop(0, n_groups, chunk_body, 0)

        for b in range(2):
            base = (2 * (n_groups - 1) + b) * chunk
            pltpu.make_async_copy(
                rows[b], out_hbm.at[pl.ds(tile_base + base, chunk)],
                sems_o[b]).wait()

    return k


def kernel(edge_attr, W0, W1, W2):
    E = edge_attr.shape[0]
    attr = edge_attr.astype(jnp.int32).T.reshape(-1)
    t = _build_table(W0, W1, W2)
    return _make_sc_kernel(E)(attr, t)


# speedup vs baseline: 15.5246x; 1.0002x over previous
import functools

import jax
import jax.numpy as jnp
from jax import lax
from jax.experimental import pallas as pl
from jax.experimental.pallas import tpu as pltpu
from jax.experimental.pallas import tpu_sc as plsc

EMB = 128
NC, NS = 2, 16
NW = NC * NS


def _table_body(w0_ref, w1_ref, w2_ref, t_ref):
    c = lax.broadcasted_iota(jnp.int32, (64, 1), 0)
    i0, r = c // 12, c % 12
    i1, i2 = r // 2, r % 2

    def pick(idx, w_ref):
        acc = jnp.zeros((64, EMB), jnp.float32)
        for row in range(w_ref.shape[0]):
            acc = acc + (idx == row).astype(jnp.float32) * w_ref[row:row + 1, :]
        return acc

    t_ref[...] = (pick(i0, w0_ref) + pick(i1, w1_ref) + pick(i2, w2_ref))


def _build_table(W0, W1, W2):
    def pad8(w):
        return jnp.zeros((8, EMB), jnp.float32).at[:w.shape[0]].set(w)

    return pl.pallas_call(
        _table_body,
        out_shape=jax.ShapeDtypeStruct((64, EMB), jnp.float32),
    )(pad8(W0), pad8(W1), pad8(W2))


def _make_sc_kernel(E):
    per_w = E // NW
    chunk = 200
    sub = 40
    nsub = chunk // sub
    n_groups = per_w // (2 * chunk)
    assert E % (NW * 2 * chunk) == 0
    mesh = plsc.VectorSubcoreMesh(core_axis_name="c", subcore_axis_name="s")

    @functools.partial(
        pl.kernel, mesh=mesh,
        out_type=jax.ShapeDtypeStruct((E, EMB), jnp.float32),
        scratch_types=[
            pltpu.VMEM_SHARED((64, EMB), jnp.float32),
            pltpu.VMEM((per_w,), jnp.int32),
            pltpu.VMEM((per_w,), jnp.int32),
            pltpu.VMEM((per_w,), jnp.int32),
            pltpu.VMEM((per_w,), jnp.int32),
            pltpu.VMEM((chunk, EMB), jnp.float32),
            pltpu.VMEM((chunk, EMB), jnp.float32),
            pltpu.SemaphoreType.DMA,
            pltpu.SemaphoreType.DMA,
            pltpu.SemaphoreType.DMA,
        ])
    def k(attr_hbm, t_hbm, out_hbm, t_s, a0_v, a1_v, a2_v, codes_v, rows0,
          rows1, sem_g, sem_o0, sem_o1):
        wid = lax.axis_index("s") * NC + lax.axis_index("c")
        tile_base = wid * per_w
        @pl.when(lax.axis_index("s") == 0)
        def _():
            pltpu.sync_copy(t_hbm, t_s)
        plsc.subcore_barrier()

        hs_a = [pltpu.async_copy(
                    attr_hbm.at[pl.ds(c * E + tile_base, per_w)], av, sem_g)
                for c, av in enumerate((a0_v, a1_v, a2_v))]
        for h in hs_a:
            h.wait()

        rows = (rows0, rows1)
        sems_o = (sem_o0, sem_o1)

        def chunk_body(g, carry):
            pair_base = 2 * g * chunk
            for j in range(2 * chunk // 16):
                s = pl.ds(pair_base + j * 16, 16)
                codes_v[s] = a0_v[s] * 12 + a1_v[s] * 2 + a2_v[s]
            for b in range(2):
                base = (2 * g + b) * chunk
                rb, so = rows[b], sems_o[b]

                @pl.when(g >= 1)
                def _():
                    pltpu.make_async_copy(
                        rb, out_hbm.at[pl.ds(tile_base + base, chunk)],
                        so).wait()

                hs = [pltpu.async_copy(
                          t_s.at[codes_v.at[pl.ds(base + j * sub, sub)]],
                          rb.at[pl.ds(j * sub, sub)], sem_g)
                      for j in range(nsub)]
                for h in hs:
                    h.wait()
                pltpu.async_copy(
                    rb, out_hbm.at[pl.ds(tile_base + base, chunk)], so)
            return carry

        lax.fori_loop(0, n_groups, chunk_body, 0)

        for b in range(2):
            base = (2 * (n_groups - 1) + b) * chunk
            pltpu.make_async_copy(
                rows[b], out_hbm.at[pl.ds(tile_base + base, chunk)],
                sems_o[b]).wait()

    return k


def kernel(edge_attr, W0, W1, W2):
    E = edge_attr.shape[0]
    attr = edge_attr.astype(jnp.int32).T.reshape(-1)
    t = _build_table(W0, W1, W2)
    return _make_sc_kernel(E)(attr, t)
